# Initial kernel scaffold; baseline (speedup 1.0000x reference)
import jax
import jax.numpy as jnp
from jax.experimental import pallas as pl


def kernel(x, edge_index, edge_attr, u, batch, edge_params, node_params, global_params):
    raise NotImplementedError("write your pallas kernel here")


# speedup vs baseline: 3.1670x; 3.1670x over previous
import functools

import jax
import jax.numpy as jnp
from jax import lax
from jax.experimental import pallas as pl
from jax.experimental.pallas import tpu as pltpu
from jax.experimental.pallas import tpu_sc as plsc

F32 = jnp.float32

_NC = 2
_NS = 16
_NW = _NC * _NS
_C = 80


def _ln(h, g, b):
    mu = jnp.mean(h, axis=-1, keepdims=True)
    d = h - mu
    var = jnp.mean(d * d, axis=-1, keepdims=True)
    return d * lax.rsqrt(var + 1e-5) * g + b


def _pre_body(x_ref, w1a_ref, w1b_ref, w1d_ref, u_ref, b1_ref, xa_ref, xb_ref):
    xblk = x_ref[...]
    c = jnp.dot(u_ref[...], w1d_ref[...], preferred_element_type=F32) + b1_ref[...]
    xa_ref[...] = jnp.dot(xblk, w1a_ref[...], preferred_element_type=F32) + c
    xb_ref[...] = jnp.dot(xblk, w1b_ref[...], preferred_element_type=F32)


def _make_gather(n_edges, d):
    ew = n_edges // _NW
    nchunk = ew // _C
    mesh = plsc.VectorSubcoreMesh(core_axis_name="c", subcore_axis_name="s",
                                  num_cores=_NC, num_subcores=_NS)

    @functools.partial(
        pl.kernel,
        mesh=mesh,
        out_type=jax.ShapeDtypeStruct((n_edges, d), F32),
        scratch_types=[
            pltpu.VMEM((_C,), jnp.int32),
            pltpu.VMEM((_C,), jnp.int32),
            pltpu.VMEM((_C, d), F32),
            pltpu.VMEM((_C, d), F32),
            pltpu.SemaphoreType.DMA,
            pltpu.SemaphoreType.DMA,
        ],
    )
    def k(xa_hbm, xb_hbm, row_hbm, col_hbm, g_hbm, idxa, idxb, bufa, bufb,
          sema, semb):
        wid = lax.axis_index("s") * _NC + lax.axis_index("c")
        base = wid * ew

        def chunk(kk, carry):
            off = base + kk * _C
            pltpu.sync_copy(row_hbm.at[pl.ds(off, _C)], idxa)
            pltpu.sync_copy(col_hbm.at[pl.ds(off, _C)], idxb)
            cpa = pltpu.async_copy(xa_hbm.at[idxa], bufa, sema)
            cpb = pltpu.async_copy(xb_hbm.at[idxb], bufb, semb)
            cpa.wait()
            cpb.wait()

            def add_row(r, c2):
                for j in range(d // 16):
                    sl = pl.ds(j * 16, 16)
                    bufa[r, sl] = bufa[r, sl] + bufb[r, sl]
                return c2

            lax.fori_loop(0, _C, add_row, 0)
            pltpu.sync_copy(bufa, g_hbm.at[pl.ds(off, _C)])
            return carry

        lax.fori_loop(0, nchunk, chunk, 0)

    return k


def _edge_body(g_ref, ea_ref, w1c_ref, g1_ref, be1_ref, w2_ref, b2_ref,
               g2_ref, be2_ref, w3_ref, b3_ref, eo_ref, esum_ref):
    h = g_ref[...] + jnp.dot(ea_ref[...], w1c_ref[...],
                             preferred_element_type=F32)
    h = jnp.maximum(h, 0.0)
    h = _ln(h, g1_ref[...], be1_ref[...])
    h = jnp.maximum(
        jnp.dot(h, w2_ref[...], preferred_element_type=F32) + b2_ref[...], 0.0)
    h = _ln(h, g2_ref[...], be2_ref[...])
    eo = jnp.dot(h, w3_ref[...], preferred_element_type=F32) + b3_ref[...]
    eo_ref[...] = eo

    @pl.when(pl.program_id(0) == 0)
    def _():
        esum_ref[...] = jnp.zeros_like(esum_ref)

    esum_ref[...] += jnp.sum(eo, axis=0, keepdims=True)


def _make_scatter(n_edges, n_nodes, d):
    ew = n_edges // _NW
    nchunk = ew // _C
    n_pad = ((n_nodes + _NS * 8 - 1) // (_NS * 8)) * (_NS * 8)
    rows_t = n_pad // _NS
    zr = 128
    nz = rows_t // zr
    mesh = plsc.VectorSubcoreMesh(core_axis_name="c", subcore_axis_name="s",
                                  num_cores=_NC, num_subcores=_NS)

    @functools.partial(
        pl.kernel,
        mesh=mesh,
        out_type=jax.ShapeDtypeStruct((_NC, n_pad, d), F32),
        scratch_types=[
            pltpu.VMEM((_C,), jnp.int32),
            pltpu.VMEM((_C, d), F32),
            pltpu.VMEM((zr, d), F32),
            pltpu.VMEM_SHARED((n_pad, d), F32),
            pltpu.SemaphoreType.DMA,
        ],
    )
    def k(eo_hbm, col_hbm, agg_hbm, idx, buf, zbuf, shared, sem):
        cid = lax.axis_index("c")
        sid = lax.axis_index("s")
        wid = sid * _NC + cid

        def zrow(r, carry):
            for j in range(d // 16):
                zbuf[r, pl.ds(j * 16, 16)] = jnp.zeros((16,), F32)
            return carry

        lax.fori_loop(0, zr, zrow, 0)
        for m in range(nz):
            pltpu.sync_copy(zbuf, shared.at[pl.ds(sid * rows_t + m * zr, zr)])
        plsc.subcore_barrier()

        base = wid * ew

        def chunk(kk, carry):
            off = base + kk * _C
            pltpu.sync_copy(col_hbm.at[pl.ds(off, _C)], idx)
            pltpu.sync_copy(eo_hbm.at[pl.ds(off, _C)], buf)
            pltpu.sync_copy(buf, shared.at[idx], add=True)
            return carry

        lax.fori_loop(0, nchunk, chunk, 0)
        plsc.subcore_barrier()
        pltpu.sync_copy(shared.at[pl.ds(sid * rows_t, rows_t)],
                        agg_hbm.at[cid, pl.ds(sid * rows_t, rows_t)])

    return k


def _node_body(x_ref, a0_ref, a1_ref, u_ref, esum_ref,
               v1x, v1a, v1u, nb1, ng1, nbe1, nw2, nb2, ng2, nbe2, nw3, nb3,
               gu, gn, ge, gb1, gg1, gbe1, gw2, gb2, gg2, gbe2, gw3, gb3,
               xnew_ref, unew_ref, nsum_ref):
    i = pl.program_id(0)
    agg = a0_ref[...] + a1_ref[...]
    h = (jnp.dot(x_ref[...], v1x[...], preferred_element_type=F32)
         + jnp.dot(agg, v1a[...], preferred_element_type=F32)
         + jnp.dot(u_ref[...], v1u[...], preferred_element_type=F32)
         + nb1[...])
    h = jnp.maximum(h, 0.0)
    h = _ln(h, ng1[...], nbe1[...])
    h = jnp.maximum(
        jnp.dot(h, nw2[...], preferred_element_type=F32) + nb2[...], 0.0)
    h = _ln(h, ng2[...], nbe2[...])
    xn = jnp.dot(h, nw3[...], preferred_element_type=F32) + nb3[...]
    xnew_ref[...] = xn

    @pl.when(i == 0)
    def _():
        nsum_ref[...] = jnp.zeros_like(nsum_ref)

    nsum_ref[...] += jnp.sum(xn, axis=0, keepdims=True)

    @pl.when(i == pl.num_programs(0) - 1)
    def _():
        gh = (jnp.dot(u_ref[...], gu[...], preferred_element_type=F32)
              + jnp.dot(nsum_ref[...], gn[...], preferred_element_type=F32)
              + jnp.dot(esum_ref[...], ge[...], preferred_element_type=F32)
              + gb1[...])
        gh = jnp.maximum(gh, 0.0)
        gh = _ln(gh, gg1[...], gbe1[...])
        gh = jnp.maximum(
            jnp.dot(gh, gw2[...], preferred_element_type=F32) + gb2[...], 0.0)
        gh = _ln(gh, gg2[...], gbe2[...])
        unew_ref[...] = jnp.dot(gh, gw3[...], preferred_element_type=F32) + gb3[...]


def _full(shape):
    return pl.BlockSpec(shape, lambda i: tuple(0 for _ in shape))


def kernel(x, edge_index, edge_attr, u, batch, edge_params, node_params,
           global_params):
    n_nodes, d = x.shape
    n_edges, d_e = edge_attr.shape
    d_u = u.shape[1]

    W1, b1, g1, be1, W2, b2, g2, be2, W3, b3 = edge_params
    V1, nb1, ng1, nbe1, V2, nb2, ng2, nbe2, V3, nb3 = node_params
    G1, gb1, gg1, gbe1, G2, gb2, gg2, gbe2, G3, gb3 = global_params

    row = edge_index[0].astype(jnp.int32)
    col = edge_index[1].astype(jnp.int32)

    W1a, W1b, W1c, W1d = W1[:d], W1[d:2 * d], W1[2 * d:2 * d + d_e], W1[2 * d + d_e:]
    V1x, V1a, V1u = V1[:d], V1[d:2 * d], V1[2 * d:]
    Gu, Gn, Ge = G1[:d_u], G1[d_u:d_u + d], G1[d_u + d:]

    r2 = lambda v: v.reshape(1, -1)

    nb_blk = 1000
    n_grid = n_nodes // nb_blk
    xa, xb = pl.pallas_call(
        _pre_body,
        grid=(n_grid,),
        in_specs=[
            pl.BlockSpec((nb_blk, d), lambda i: (i, 0)),
            _full((d, d)), _full((d, d)), _full((d_u, d)),
            _full((1, d_u)), _full((1, d)),
        ],
        out_specs=[
            pl.BlockSpec((nb_blk, d), lambda i: (i, 0)),
            pl.BlockSpec((nb_blk, d), lambda i: (i, 0)),
        ],
        out_shape=[
            jax.ShapeDtypeStruct((n_nodes, d), F32),
            jax.ShapeDtypeStruct((n_nodes, d), F32),
        ],
    )(x, W1a, W1b, W1d, u, r2(b1))

    g = _make_gather(n_edges, d)(xa, xb, row, col)

    eb_blk = 4000
    e_grid = n_edges // eb_blk
    edge_out, esum = pl.pallas_call(
        _edge_body,
        grid=(e_grid,),
        in_specs=[
            pl.BlockSpec((eb_blk, d), lambda i: (i, 0)),
            pl.BlockSpec((eb_blk, d_e), lambda i: (i, 0)),
            _full((d_e, d)), _full((1, d)), _full((1, d)),
            _full((d, d)), _full((1, d)), _full((1, d)), _full((1, d)),
            _full((d, d)), _full((1, d)),
        ],
        out_specs=[
            pl.BlockSpec((eb_blk, d), lambda i: (i, 0)),
            pl.BlockSpec((1, d), lambda i: (0, 0)),
        ],
        out_shape=[
            jax.ShapeDtypeStruct((n_edges, d), F32),
            jax.ShapeDtypeStruct((1, d), F32),
        ],
    )(g, edge_attr, W1c, r2(g1), r2(be1), W2, r2(b2), r2(g2), r2(be2),
      W3, r2(b3))

    aggp = _make_scatter(n_edges, n_nodes, d)(edge_out, col)

    x_new, u_new = pl.pallas_call(
        _node_body,
        grid=(n_grid,),
        in_specs=[
            pl.BlockSpec((nb_blk, d), lambda i: (i, 0)),
            pl.BlockSpec((nb_blk, d), lambda i: (i, 0)),
            pl.BlockSpec((nb_blk, d), lambda i: (i, 0)),
            _full((1, d_u)), _full((1, d)),
            _full((d, d)), _full((d, d)), _full((d_u, d)),
            _full((1, d)), _full((1, d)), _full((1, d)),
            _full((d, d)), _full((1, d)), _full((1, d)), _full((1, d)),
            _full((d, d)), _full((1, d)),
            _full((d_u, d)), _full((d, d)), _full((d, d)),
            _full((1, d)), _full((1, d)), _full((1, d)),
            _full((d, d)), _full((1, d)), _full((1, d)), _full((1, d)),
            _full((d, 1)), _full((1, 1)),
        ],
        out_specs=[
            pl.BlockSpec((nb_blk, d), lambda i: (i, 0)),
            pl.BlockSpec((1, 1), lambda i: (0, 0)),
        ],
        out_shape=[
            jax.ShapeDtypeStruct((n_nodes, d), F32),
            jax.ShapeDtypeStruct((1, 1), F32),
        ],
        scratch_shapes=[pltpu.VMEM((1, d), F32)],
    )(x, aggp[0], aggp[1], u, esum,
      V1x, V1a, V1u, r2(nb1), r2(ng1), r2(nbe1), V2, r2(nb2), r2(ng2),
      r2(nbe2), V3, r2(nb3),
      Gu, Gn, Ge, r2(gb1), r2(gg1), r2(gbe1), G2, r2(gb2), r2(gg2),
      r2(gbe2), G3, r2(gb3))

    return (x_new, edge_out, u_new)


# speedup vs baseline: 4.5113x; 1.4245x over previous
import functools

import jax
import jax.numpy as jnp
from jax import lax
from jax.experimental import pallas as pl
from jax.experimental.pallas import tpu as pltpu
from jax.experimental.pallas import tpu_sc as plsc

F32 = jnp.float32

_NC = 2
_NS = 16
_NW = _NC * _NS
_C = 80


def _ln(h, g, b):
    mu = jnp.mean(h, axis=-1, keepdims=True)
    d = h - mu
    var = jnp.mean(d * d, axis=-1, keepdims=True)
    return d * lax.rsqrt(var + 1e-5) * g + b


def _pre_body(x_ref, w1a_ref, w1b_ref, w1d_ref, u_ref, b1_ref, xa_ref, xb_ref):
    xblk = x_ref[...]
    c = jnp.dot(u_ref[...], w1d_ref[...], preferred_element_type=F32) + b1_ref[...]
    xa_ref[...] = jnp.dot(xblk, w1a_ref[...], preferred_element_type=F32) + c
    xb_ref[...] = jnp.dot(xblk, w1b_ref[...], preferred_element_type=F32)


def _make_gather(n_edges, d):
    ew = n_edges // _NW
    nchunk = ew // _C
    mesh = plsc.VectorSubcoreMesh(core_axis_name="c", subcore_axis_name="s",
                                  num_cores=_NC, num_subcores=_NS)

    @functools.partial(
        pl.kernel,
        mesh=mesh,
        out_type=jax.ShapeDtypeStruct((n_edges, d), F32),
        scratch_types=[
            pltpu.VMEM((ew,), jnp.int32),
            pltpu.VMEM((ew,), jnp.int32),
            [pltpu.VMEM((_C, d), F32) for _ in range(2)],
            [pltpu.VMEM((_C, d), F32) for _ in range(2)],
            [pltpu.SemaphoreType.DMA for _ in range(2)],
            [pltpu.SemaphoreType.DMA for _ in range(2)],
            [pltpu.SemaphoreType.DMA for _ in range(2)],
        ],
    )
    def k(xa_hbm, xb_hbm, row_hbm, col_hbm, g_hbm, rows, cols, bufa, bufb,
          sga, sgb, sst):
        wid = lax.axis_index("s") * _NC + lax.axis_index("c")
        base = wid * ew
        pltpu.sync_copy(row_hbm.at[pl.ds(base, ew)], rows)
        pltpu.sync_copy(col_hbm.at[pl.ds(base, ew)], cols)

        def fire(kk, par):
            sl = pl.ds(kk * _C, _C)
            pltpu.async_copy(xa_hbm.at[rows.at[sl]], bufa[par], sga[par])
            pltpu.async_copy(xb_hbm.at[cols.at[sl]], bufb[par], sgb[par])

        def drain_g(par):
            pltpu.make_async_copy(xa_hbm.at[pl.ds(0, _C)], bufa[par],
                                  sga[par]).wait()
            pltpu.make_async_copy(xb_hbm.at[pl.ds(0, _C)], bufb[par],
                                  sgb[par]).wait()

        def drain_st(par):
            pltpu.make_async_copy(bufa[par], g_hbm.at[pl.ds(0, _C)],
                                  sst[par]).wait()

        def add(par):
            ba, bb = bufa[par], bufb[par]

            def add_row(r, c2):
                for j in range(d // 16):
                    sl = pl.ds(j * 16, 16)
                    ba[r, sl] = ba[r, sl] + bb[r, sl]
                return c2

            lax.fori_loop(0, _C, add_row, 0)

        fire(0, 0)

        def body(i, carry):
            for par in range(2):
                kk = 2 * i + par
                drain_g(par)

                @pl.when(kk + 1 < nchunk)
                def _():
                    @pl.when(kk >= 1)
                    def _():
                        drain_st(1 - par)

                    fire(kk + 1, 1 - par)

                add(par)
                pltpu.async_copy(bufa[par], g_hbm.at[pl.ds(base + kk * _C, _C)],
                                 sst[par])
            return carry

        lax.fori_loop(0, nchunk // 2, body, 0)

        @pl.when((nchunk % 2) == 1)
        def _():
            par = (nchunk - 1) % 2
            drain_g(par)
            add(par)
            pltpu.async_copy(bufa[par],
                             g_hbm.at[pl.ds(base + (nchunk - 1) * _C, _C)],
                             sst[par])

        drain_st(0)
        drain_st(1)

    return k


def _edge_body(g_ref, ea_ref, w1c_ref, g1_ref, be1_ref, w2_ref, b2_ref,
               g2_ref, be2_ref, w3_ref, b3_ref, eo_ref, esum_ref):
    h = g_ref[...] + jnp.dot(ea_ref[...], w1c_ref[...],
                             preferred_element_type=F32)
    h = jnp.maximum(h, 0.0)
    h = _ln(h, g1_ref[...], be1_ref[...])
    h = jnp.maximum(
        jnp.dot(h, w2_ref[...], preferred_element_type=F32) + b2_ref[...], 0.0)
    h = _ln(h, g2_ref[...], be2_ref[...])
    eo = jnp.dot(h, w3_ref[...], preferred_element_type=F32) + b3_ref[...]
    eo_ref[...] = eo

    @pl.when(pl.program_id(0) == 0)
    def _():
        esum_ref[...] = jnp.zeros_like(esum_ref)

    esum_ref[...] += jnp.sum(eo, axis=0, keepdims=True)


def _make_scatter(n_edges, n_nodes, d):
    ew = n_edges // _NW
    nchunk = ew // _C
    n_pad = ((n_nodes + _NS * 8 - 1) // (_NS * 8)) * (_NS * 8)
    rows_t = n_pad // _NS
    zr = 128
    nz = rows_t // zr
    mesh = plsc.VectorSubcoreMesh(core_axis_name="c", subcore_axis_name="s",
                                  num_cores=_NC, num_subcores=_NS)

    @functools.partial(
        pl.kernel,
        mesh=mesh,
        out_type=jax.ShapeDtypeStruct((_NC, n_pad, d), F32),
        scratch_types=[
            [pltpu.VMEM((_C,), jnp.int32) for _ in range(2)],
            [pltpu.VMEM((_C, d), F32) for _ in range(2)],
            pltpu.VMEM((zr, d), F32),
            pltpu.VMEM_SHARED((n_pad, d), F32),
            [pltpu.SemaphoreType.DMA for _ in range(2)],
            [pltpu.SemaphoreType.DMA for _ in range(2)],
        ],
    )
    def k(eo_hbm, col_hbm, agg_hbm, idx, buf, zbuf, shared, sidx, sdat):
        cid = lax.axis_index("c")
        sid = lax.axis_index("s")
        wid = sid * _NC + cid
        base = wid * ew

        def fire(kk, par):
            off = base + kk * _C
            pltpu.async_copy(col_hbm.at[pl.ds(off, _C)], idx[par], sidx[par])
            pltpu.async_copy(eo_hbm.at[pl.ds(off, _C)], buf[par], sdat[par])

        def drain_in(par):
            pltpu.make_async_copy(col_hbm.at[pl.ds(0, _C)], idx[par],
                                  sidx[par]).wait()
            pltpu.make_async_copy(eo_hbm.at[pl.ds(0, _C)], buf[par],
                                  sdat[par]).wait()

        fire(0, 0)

        def zrow(r, carry):
            for j in range(d // 16):
                zbuf[r, pl.ds(j * 16, 16)] = jnp.zeros((16,), F32)
            return carry

        lax.fori_loop(0, zr, zrow, 0)
        for m in range(nz):
            pltpu.sync_copy(zbuf, shared.at[pl.ds(sid * rows_t + m * zr, zr)])
        plsc.subcore_barrier()

        def body(i, carry):
            for par in range(2):
                kk = 2 * i + par
                drain_in(par)

                @pl.when(kk + 1 < nchunk)
                def _():
                    fire(kk + 1, 1 - par)

                pltpu.sync_copy(buf[par], shared.at[idx[par]], add=True)
            return carry

        lax.fori_loop(0, nchunk // 2, body, 0)

        @pl.when((nchunk % 2) == 1)
        def _():
            par = (nchunk - 1) % 2
            drain_in(par)
            pltpu.sync_copy(buf[par], shared.at[idx[par]], add=True)

        plsc.subcore_barrier()
        pltpu.sync_copy(shared.at[pl.ds(sid * rows_t, rows_t)],
                        agg_hbm.at[cid, pl.ds(sid * rows_t, rows_t)])

    return k


def _node_body(x_ref, a0_ref, a1_ref, u_ref, esum_ref,
               v1x, v1a, v1u, nb1, ng1, nbe1, nw2, nb2, ng2, nbe2, nw3, nb3,
               gu, gn, ge, gb1, gg1, gbe1, gw2, gb2, gg2, gbe2, gw3, gb3,
               xnew_ref, unew_ref, nsum_ref):
    i = pl.program_id(0)
    agg = a0_ref[...] + a1_ref[...]
    h = (jnp.dot(x_ref[...], v1x[...], preferred_element_type=F32)
         + jnp.dot(agg, v1a[...], preferred_element_type=F32)
         + jnp.dot(u_ref[...], v1u[...], preferred_element_type=F32)
         + nb1[...])
    h = jnp.maximum(h, 0.0)
    h = _ln(h, ng1[...], nbe1[...])
    h = jnp.maximum(
        jnp.dot(h, nw2[...], preferred_element_type=F32) + nb2[...], 0.0)
    h = _ln(h, ng2[...], nbe2[...])
    xn = jnp.dot(h, nw3[...], preferred_element_type=F32) + nb3[...]
    xnew_ref[...] = xn

    @pl.when(i == 0)
    def _():
        nsum_ref[...] = jnp.zeros_like(nsum_ref)

    nsum_ref[...] += jnp.sum(xn, axis=0, keepdims=True)

    @pl.when(i == pl.num_programs(0) - 1)
    def _():
        gh = (jnp.dot(u_ref[...], gu[...], preferred_element_type=F32)
              + jnp.dot(nsum_ref[...], gn[...], preferred_element_type=F32)
              + jnp.dot(esum_ref[...], ge[...], preferred_element_type=F32)
              + gb1[...])
        gh = jnp.maximum(gh, 0.0)
        gh = _ln(gh, gg1[...], gbe1[...])
        gh = jnp.maximum(
            jnp.dot(gh, gw2[...], preferred_element_type=F32) + gb2[...], 0.0)
        gh = _ln(gh, gg2[...], gbe2[...])
        unew_ref[...] = jnp.dot(gh, gw3[...], preferred_element_type=F32) + gb3[...]


def _full(shape):
    return pl.BlockSpec(shape, lambda i: tuple(0 for _ in shape))


def kernel(x, edge_index, edge_attr, u, batch, edge_params, node_params,
           global_params):
    n_nodes, d = x.shape
    n_edges, d_e = edge_attr.shape
    d_u = u.shape[1]

    W1, b1, g1, be1, W2, b2, g2, be2, W3, b3 = edge_params
    V1, nb1, ng1, nbe1, V2, nb2, ng2, nbe2, V3, nb3 = node_params
    G1, gb1, gg1, gbe1, G2, gb2, gg2, gbe2, G3, gb3 = global_params

    row = edge_index[0].astype(jnp.int32)
    col = edge_index[1].astype(jnp.int32)

    W1a, W1b, W1c, W1d = W1[:d], W1[d:2 * d], W1[2 * d:2 * d + d_e], W1[2 * d + d_e:]
    V1x, V1a, V1u = V1[:d], V1[d:2 * d], V1[2 * d:]
    Gu, Gn, Ge = G1[:d_u], G1[d_u:d_u + d], G1[d_u + d:]

    r2 = lambda v: v.reshape(1, -1)

    nb_blk = 1000
    n_grid = n_nodes // nb_blk
    xa, xb = pl.pallas_call(
        _pre_body,
        grid=(n_grid,),
        in_specs=[
            pl.BlockSpec((nb_blk, d), lambda i: (i, 0)),
            _full((d, d)), _full((d, d)), _full((d_u, d)),
            _full((1, d_u)), _full((1, d)),
        ],
        out_specs=[
            pl.BlockSpec((nb_blk, d), lambda i: (i, 0)),
            pl.BlockSpec((nb_blk, d), lambda i: (i, 0)),
        ],
        out_shape=[
            jax.ShapeDtypeStruct((n_nodes, d), F32),
            jax.ShapeDtypeStruct((n_nodes, d), F32),
        ],
    )(x, W1a, W1b, W1d, u, r2(b1))

    g = _make_gather(n_edges, d)(xa, xb, row, col)

    eb_blk = 4000
    e_grid = n_edges // eb_blk
    edge_out, esum = pl.pallas_call(
        _edge_body,
        grid=(e_grid,),
        in_specs=[
            pl.BlockSpec((eb_blk, d), lambda i: (i, 0)),
            pl.BlockSpec((eb_blk, d_e), lambda i: (i, 0)),
            _full((d_e, d)), _full((1, d)), _full((1, d)),
            _full((d, d)), _full((1, d)), _full((1, d)), _full((1, d)),
            _full((d, d)), _full((1, d)),
        ],
        out_specs=[
            pl.BlockSpec((eb_blk, d), lambda i: (i, 0)),
            pl.BlockSpec((1, d), lambda i: (0, 0)),
        ],
        out_shape=[
            jax.ShapeDtypeStruct((n_edges, d), F32),
            jax.ShapeDtypeStruct((1, d), F32),
        ],
    )(g, edge_attr, W1c, r2(g1), r2(be1), W2, r2(b2), r2(g2), r2(be2),
      W3, r2(b3))

    aggp = _make_scatter(n_edges, n_nodes, d)(edge_out, col)

    x_new, u_new = pl.pallas_call(
        _node_body,
        grid=(n_grid,),
        in_specs=[
            pl.BlockSpec((nb_blk, d), lambda i: (i, 0)),
            pl.BlockSpec((nb_blk, d), lambda i: (i, 0)),
            pl.BlockSpec((nb_blk, d), lambda i: (i, 0)),
            _full((1, d_u)), _full((1, d)),
            _full((d, d)), _full((d, d)), _full((d_u, d)),
            _full((1, d)), _full((1, d)), _full((1, d)),
            _full((d, d)), _full((1, d)), _full((1, d)), _full((1, d)),
            _full((d, d)), _full((1, d)),
            _full((d_u, d)), _full((d, d)), _full((d, d)),
            _full((1, d)), _full((1, d)), _full((1, d)),
            _full((d, d)), _full((1, d)), _full((1, d)), _full((1, d)),
            _full((d, 1)), _full((1, 1)),
        ],
        out_specs=[
            pl.BlockSpec((nb_blk, d), lambda i: (i, 0)),
            pl.BlockSpec((1, 1), lambda i: (0, 0)),
        ],
        out_shape=[
            jax.ShapeDtypeStruct((n_nodes, d), F32),
            jax.ShapeDtypeStruct((1, 1), F32),
        ],
        scratch_shapes=[pltpu.VMEM((1, d), F32)],
    )(x, aggp[0], aggp[1], u, esum,
      V1x, V1a, V1u, r2(nb1), r2(ng1), r2(nbe1), V2, r2(nb2), r2(ng2),
      r2(nbe2), V3, r2(nb3),
      Gu, Gn, Ge, r2(gb1), r2(gg1), r2(gbe1), G2, r2(gb2), r2(gg2),
      r2(gbe2), G3, r2(gb3))

    return (x_new, edge_out, u_new)
